# Initial kernel scaffold; baseline (speedup 1.0000x reference)
import jax
import jax.numpy as jnp
from jax.experimental import pallas as pl

_N_EMB = 1024
_E_DIM = 32
_TILE = 512


def _vq_kernel(z_ref, w_ref, dist_ref, idx_ref):
    zb = z_ref[0]
    w = w_ref[...]
    zt = zb.T
    ez = jax.lax.dot_general(
        zt, w, (((1,), (1,)), ((), ())),
        preferred_element_type=jnp.float32)
    zsq = jnp.sum(zb * zb, axis=0)[:, None]
    esq = jnp.sum(w * w, axis=1)[None, :]
    dist = zsq + esq - 2.0 * ez
    dist_ref[...] = dist
    mins = jnp.min(dist, axis=1, keepdims=True)
    lane = jax.lax.broadcasted_iota(jnp.int32, dist.shape, 1)
    idx = jnp.min(jnp.where(dist == mins, lane, _N_EMB), axis=1)
    idx_ref[0] = idx.astype(jnp.int32)


def kernel(z, emb_weight):
    bs, c, h, w = z.shape
    hw = h * w
    n = bs * hw
    blocks_per_batch = hw // _TILE
    z3 = z.reshape(bs, c, hw)
    grid = (bs, blocks_per_batch)
    dist, idx = pl.pallas_call(
        _vq_kernel,
        grid=grid,
        in_specs=[
            pl.BlockSpec((1, c, _TILE), lambda b, t: (b, 0, t)),
            pl.BlockSpec((_N_EMB, _E_DIM), lambda b, t: (0, 0)),
        ],
        out_specs=[
            pl.BlockSpec((_TILE, _N_EMB),
                         lambda b, t, bpb=blocks_per_batch: (b * bpb + t, 0)),
            pl.BlockSpec((1, _TILE),
                         lambda b, t, bpb=blocks_per_batch: (b * bpb + t, 0)),
        ],
        out_shape=[
            jax.ShapeDtypeStruct((n, _N_EMB), jnp.float32),
            jax.ShapeDtypeStruct((n // _TILE, _TILE), jnp.int32),
        ],
    )(z3, emb_weight)
    return dist, idx.reshape(n)


# speedup vs baseline: 1.6645x; 1.6645x over previous
import jax
import jax.numpy as jnp
from jax.experimental import pallas as pl

_N_EMB = 1024
_E_DIM = 32
_TILE = 512


def _vq_kernel(z_ref, w_ref, dist_ref, idx_ref):
    zb = z_ref[0]
    w = w_ref[...]
    zt = zb.T
    ez = jax.lax.dot_general(
        zt, w, (((1,), (1,)), ((), ())),
        preferred_element_type=jnp.float32)
    zsq = jnp.sum(zb * zb, axis=0)[:, None]
    esq = jnp.sum(w * w, axis=1)[None, :]
    dist = zsq + esq - 2.0 * ez
    dist_ref[...] = dist
    mins = jnp.min(dist, axis=1, keepdims=True)
    lane = jax.lax.broadcasted_iota(jnp.int32, dist.shape, 1)
    idx = jnp.min(jnp.where(dist == mins, lane, _N_EMB), axis=1)
    idx_ref[0, 0] = idx.astype(jnp.int32)


def kernel(z, emb_weight):
    bs, c, h, w = z.shape
    hw = h * w
    n = bs * hw
    blocks_per_batch = hw // _TILE
    z3 = z.reshape(bs, c, hw)
    grid = (bs, blocks_per_batch)
    dist, idx = pl.pallas_call(
        _vq_kernel,
        grid=grid,
        in_specs=[
            pl.BlockSpec((1, c, _TILE), lambda b, t: (b, 0, t)),
            pl.BlockSpec((_N_EMB, _E_DIM), lambda b, t: (0, 0)),
        ],
        out_specs=[
            pl.BlockSpec((_TILE, _N_EMB),
                         lambda b, t, bpb=blocks_per_batch: (b * bpb + t, 0)),
            pl.BlockSpec((1, 1, _TILE),
                         lambda b, t, bpb=blocks_per_batch: (b * bpb + t, 0, 0)),
        ],
        out_shape=[
            jax.ShapeDtypeStruct((n, _N_EMB), jnp.float32),
            jax.ShapeDtypeStruct((n // _TILE, 1, _TILE), jnp.int32),
        ],
    )(z3, emb_weight)
    return dist, idx.reshape(n)


# speedup vs baseline: 2.2482x; 1.3507x over previous
import jax
import jax.numpy as jnp
from jax.experimental import pallas as pl
from jax.experimental.pallas import tpu as pltpu

_N_EMB = 1024
_E_DIM = 32
_TILE = 512


def _vq_kernel(z_ref, w_ref, dist_ref, idx_ref, rhs_ref):
    @pl.when(jnp.logical_and(pl.program_id(0) == 0, pl.program_id(1) == 0))
    def _init():
        w0 = w_ref[...]
        esq = jnp.sum(w0 * w0, axis=1)[:, None]
        ones = jnp.ones((_N_EMB, 1), jnp.float32)
        rhs_ref[...] = jnp.concatenate([w0 * -2.0, ones, esq], axis=1)

    zb = z_ref[0]
    zsq = jnp.sum(zb * zb, axis=0)[:, None]
    zt = zb.T
    ones_t = jnp.ones((_TILE, 1), jnp.float32)
    lhs = jnp.concatenate([zt, zsq, ones_t], axis=1)
    dist = jax.lax.dot_general(
        lhs, rhs_ref[...], (((1,), (1,)), ((), ())),
        preferred_element_type=jnp.float32)
    dist_ref[...] = dist
    mins = jnp.min(dist, axis=1, keepdims=True)
    lane = jax.lax.broadcasted_iota(
        jnp.int32, dist.shape, 1).astype(jnp.float32)
    idxf = jnp.min(jnp.where(dist == mins, lane, jnp.float32(65536.0)),
                   axis=1, keepdims=True)
    idx_ref[0] = idxf.astype(jnp.int32)


def kernel(z, emb_weight):
    bs, c, h, w = z.shape
    hw = h * w
    n = bs * hw
    blocks_per_batch = hw // _TILE
    z3 = z.reshape(bs, c, hw)
    grid = (bs, blocks_per_batch)
    dist, idx = pl.pallas_call(
        _vq_kernel,
        grid=grid,
        in_specs=[
            pl.BlockSpec((1, c, _TILE), lambda b, t: (b, 0, t)),
            pl.BlockSpec((_N_EMB, _E_DIM), lambda b, t: (0, 0)),
        ],
        out_specs=[
            pl.BlockSpec((_TILE, _N_EMB),
                         lambda b, t, bpb=blocks_per_batch: (b * bpb + t, 0)),
            pl.BlockSpec((1, _TILE, 1),
                         lambda b, t, bpb=blocks_per_batch: (b * bpb + t, 0, 0)),
        ],
        out_shape=[
            jax.ShapeDtypeStruct((n, _N_EMB), jnp.float32),
            jax.ShapeDtypeStruct((n // _TILE, _TILE, 1), jnp.int32),
        ],
        scratch_shapes=[pltpu.VMEM((_N_EMB, _E_DIM + 2), jnp.float32)],
    )(z3, emb_weight)
    return dist, idx.reshape(n)
